# baseline (speedup 1.0000x reference)
import jax
import jax.numpy as jnp
from jax import lax
from jax.experimental import pallas as pl
from jax.experimental.pallas import tpu as pltpu
from jax.experimental.pallas import tpu_sc as plsc

B, S, HIDDEN = 4, 4096, 2048
N = B * S
NC, NS = 2, 16
NW = NC * NS
TOK_PER_W = N // NW
K = 8
N_CHUNKS = TOK_PER_W // K
NBUF = 2
LANES = 16


def _sc_body(in_hbm, idx_hbm, tab_hbm, out_hbm, idx_v,
             in0, in1, pos0, pos1, res0, res1,
             sem_in0, sem_in1, sem_gat0, sem_gat1, sem_out0, sem_out1):
    in_b = (in0, in1)
    pos_b = (pos0, pos1)
    res_b = (res0, res1)
    sem_in = (sem_in0, sem_in1)
    sem_gat = (sem_gat0, sem_gat1)
    sem_out = (sem_out0, sem_out1)

    wid = lax.axis_index("s") * NC + lax.axis_index("c")
    base = wid * TOK_PER_W
    pltpu.sync_copy(idx_hbm.at[pl.ds(base, TOK_PER_W)], idx_v)

    def fetch_start(ci, b):
        pltpu.async_copy(in_hbm.at[pl.ds(base + ci * K, K)], in_b[b],
                         sem_in[b])
        pltpu.async_copy(tab_hbm.at[idx_v.at[pl.ds(ci * K, K)]], pos_b[b],
                         sem_gat[b])

    def fetch_wait(b):
        pltpu.make_async_copy(in_hbm.at[pl.ds(0, K)], in_b[b],
                              sem_in[b]).wait()
        pltpu.make_async_copy(tab_hbm.at[idx_v.at[pl.ds(0, K)]], pos_b[b],
                              sem_gat[b]).wait()

    def out_start(ci, b):
        pltpu.async_copy(res_b[b], out_hbm.at[pl.ds(base + ci * K, K)],
                         sem_out[b])

    def out_wait(b):
        pltpu.make_async_copy(res_b[b], out_hbm.at[pl.ds(0, K)],
                              sem_out[b]).wait()

    fetch_start(0, 0)
    fetch_start(1, 1)

    @pl.loop(0, N_CHUNKS, step=NBUF)
    def _chunk(c):
        for b in range(NBUF):
            ci = c + b
            fetch_wait(b)

            @pl.when(ci >= 2)
            def _res_free():
                out_wait(b)

            for h in range(2):
                @pl.loop(h * (K // 2), (h + 1) * (K // 2))
                def _row(r):
                    @plsc.parallel_loop(0, HIDDEN // LANES, unroll=8)
                    def _vec(j):
                        sl = pl.ds(j * LANES, LANES)
                        res_b[b][r, sl] = in_b[b][r, sl] + pos_b[b][r, sl]

                pltpu.async_copy(
                    res_b[b].at[pl.ds(h * (K // 2), K // 2)],
                    out_hbm.at[pl.ds(base + ci * K + h * (K // 2), K // 2)],
                    sem_out[b])

            @pl.when(ci + 2 < N_CHUNKS)
            def _prefetch():
                fetch_start(ci + 2, b)

    out_wait(0)
    out_wait(1)


def _sc_call(in_flat, idx_flat, pos_table):
    mesh = plsc.VectorSubcoreMesh(core_axis_name="c", subcore_axis_name="s",
                                  num_cores=NC, num_subcores=NS)
    return pl.kernel(
        _sc_body,
        out_type=jax.ShapeDtypeStruct((N, HIDDEN), jnp.float32),
        mesh=mesh,
        scratch_types=[
            pltpu.VMEM((TOK_PER_W,), jnp.int32),
        ] + [pltpu.VMEM((K, HIDDEN), jnp.float32)] * 6
          + [pltpu.SemaphoreType.DMA] * 6,
    )(in_flat, idx_flat, pos_table)


def kernel(input_ids, position_ids, pos_table):
    in_flat = input_ids.reshape(N, HIDDEN)
    idx_flat = position_ids.reshape(N).astype(jnp.int32)
    out = _sc_call(in_flat, idx_flat, pos_table)
    return out.reshape(B, S, HIDDEN)


# speedup vs baseline: 1.0051x; 1.0051x over previous
import jax
import jax.numpy as jnp
from jax import lax
from jax.experimental import pallas as pl
from jax.experimental.pallas import tpu as pltpu
from jax.experimental.pallas import tpu_sc as plsc

B, S, HIDDEN = 4, 4096, 2048
N = B * S
NC, NS = 2, 16
NW = NC * NS
TOK_PER_W = N // NW
K = 8
N_CHUNKS = TOK_PER_W // K
NBUF = 2
LANES = 16


def _sc_body(in_hbm, idx_hbm, tab_hbm, out_hbm, idx_v,
             in0, in1, pos0, pos1, res0, res1,
             sem_in0, sem_in1, sem_gat0, sem_gat1, sem_out0, sem_out1):
    in_b = (in0, in1)
    pos_b = (pos0, pos1)
    res_b = (res0, res1)
    sem_in = (sem_in0, sem_in1)
    sem_gat = (sem_gat0, sem_gat1)
    sem_out = (sem_out0, sem_out1)

    wid = lax.axis_index("s") * NC + lax.axis_index("c")
    base = wid * TOK_PER_W
    pltpu.sync_copy(idx_hbm.at[pl.ds(base, TOK_PER_W)], idx_v)

    def fetch_start(ci, b):
        pltpu.async_copy(in_hbm.at[pl.ds(base + ci * K, K)], in_b[b],
                         sem_in[b])
        pltpu.async_copy(tab_hbm.at[idx_v.at[pl.ds(ci * K, K)]], pos_b[b],
                         sem_gat[b])

    def fetch_wait(b):
        pltpu.make_async_copy(in_hbm.at[pl.ds(0, K)], in_b[b],
                              sem_in[b]).wait()
        pltpu.make_async_copy(tab_hbm.at[idx_v.at[pl.ds(0, K)]], pos_b[b],
                              sem_gat[b]).wait()

    def out_start(ci, b):
        pltpu.async_copy(res_b[b], out_hbm.at[pl.ds(base + ci * K, K)],
                         sem_out[b])

    def out_wait(b):
        pltpu.make_async_copy(res_b[b], out_hbm.at[pl.ds(0, K)],
                              sem_out[b]).wait()

    fetch_start(0, 0)
    fetch_start(1, 1)

    @pl.loop(0, N_CHUNKS, step=NBUF)
    def _chunk(c):
        for b in range(NBUF):
            ci = c + b
            fetch_wait(b)

            @pl.when(ci >= 2)
            def _res_free():
                out_wait(b)

            @pl.loop(0, K)
            def _row(r):
                @plsc.parallel_loop(0, HIDDEN // LANES, unroll=8)
                def _vec(j):
                    sl = pl.ds(j * LANES, LANES)
                    res_b[b][r, sl] = in_b[b][r, sl] + pos_b[b][r, sl]

            out_start(ci, b)

            @pl.when(ci + 2 < N_CHUNKS)
            def _prefetch():
                fetch_start(ci + 2, b)

    out_wait(0)
    out_wait(1)


def _sc_call(in_flat, idx_flat, pos_table):
    mesh = plsc.VectorSubcoreMesh(core_axis_name="c", subcore_axis_name="s",
                                  num_cores=NC, num_subcores=NS)
    return pl.kernel(
        _sc_body,
        out_type=jax.ShapeDtypeStruct((N, HIDDEN), jnp.float32),
        mesh=mesh,
        scratch_types=[
            pltpu.VMEM((TOK_PER_W,), jnp.int32),
        ] + [pltpu.VMEM((K, HIDDEN), jnp.float32)] * 6
          + [pltpu.SemaphoreType.DMA] * 6,
    )(in_flat, idx_flat, pos_table)


def kernel(input_ids, position_ids, pos_table):
    in_flat = input_ids.reshape(N, HIDDEN)
    idx_flat = position_ids.reshape(N).astype(jnp.int32)
    out = _sc_call(in_flat, idx_flat, pos_table)
    return out.reshape(B, S, HIDDEN)
